# baseline (speedup 1.0000x reference)
import functools

import jax
import jax.numpy as jnp
from jax import lax
from jax.experimental import pallas as pl
from jax.experimental.pallas import tpu as pltpu
from jax.experimental.pallas import tpu_sc as plsc

NC = 2
NS = 16
LN = 16
CH = 128
HH = 128
ZR = 128


def _fill(buf, rows, value):
    dt = jnp.dtype(buf.dtype)
    lanes = LN * (4 // dt.itemsize)
    ncol = buf.shape[1] // lanes
    v = jnp.full((lanes,), value, dt)
    for r in range(rows):
        for j in range(ncol):
            buf[r, pl.ds(j * lanes, lanes)] = v


def _row_split(N):
    RPT = (N // NS) // 8 * 8
    REM0 = RPT * (NS - 1)
    REM = N - REM0
    ZN = -(-max(RPT, REM) // ZR)
    assert REM0 + ZN * ZR <= N + 8 and REM >= RPT and (REM - RPT) % 8 == 0
    return RPT, REM0, REM, ZN


@functools.lru_cache(maxsize=None)
def _deg_kernel(N, EP):
    nch = EP // CH
    npt = nch // (NC * NS)
    NW = -(-N // 128) * 128
    NPAD = NW + 2 * LN
    mesh = plsc.VectorSubcoreMesh(core_axis_name="c", subcore_axis_name="s")
    ntk = NC * NS
    ne = npt * CH

    def body(dst2_hbm, out_hbm, idx2, hist):
        cid = lax.axis_index("c")
        sid = lax.axis_index("s")
        tid = cid * NS + sid
        pltpu.sync_copy(dst2_hbm.at[pl.ds(tid * npt, npt)], idx2)

        def zrow(r, _):
            hist[0, pl.ds(r * LN, LN)] = jnp.zeros((LN,), jnp.float32)
            return 0

        lax.fori_loop(0, NPAD // LN, zrow, 0)

        e0 = jnp.where(jnp.arange(LN, dtype=jnp.int32) == 0,
                       jnp.float32(1.0), jnp.float32(0.0))

        def chunk(k, _):
            for j in range(CH // LN):
                v = idx2[k, pl.ds(j * LN, LN)]
                for l in range(LN):
                    d = v[l]
                    w = hist[0, pl.ds(d, LN)]
                    hist[0, pl.ds(d, LN)] = w + e0
            return 0

        lax.fori_loop(0, npt, chunk, 0)
        pltpu.sync_copy(hist.at[:, pl.ds(0, NW)], out_hbm.at[tid])

    return pl.kernel(
        body,
        out_type=jax.ShapeDtypeStruct((ntk, 1, NW), jnp.float32),
        mesh=mesh,
        scratch_types=[
            pltpu.VMEM((npt, CH), jnp.int32),
            pltpu.VMEM((1, NPAD), jnp.float32),
        ],
    )


@functools.lru_cache(maxsize=None)
def _scatter_kernel(N, EP):
    nch = EP // CH
    npt = nch // NS
    RPT, REM0, REM, ZN = _row_split(N)
    NPAD = N + 8
    mesh = plsc.VectorSubcoreMesh(core_axis_name="c", subcore_axis_name="s")

    NSG = 2
    cps = npt // NSG
    assert npt % NSG == 0 and cps % 2 == 0 and cps % 8 == 0

    def body(yp0, yp1, src2, dst2, out0, out1,
             gidx2, sidx2, bufs, acc, sem0, sem1):
        cid = lax.axis_index("c")
        sid = lax.axis_index("s")
        b0 = bufs.at[0]
        b1 = bufs.at[1]
        _fill(b0, ZR, 0.0)
        r0 = pl.multiple_of(sid * RPT, 8)
        for j in range(ZN):
            pltpu.sync_copy(b0, acc.at[pl.ds(r0 + j * ZR, ZR)])
        c0 = sid * npt
        plsc.subcore_barrier()

        def run(yp):
            for s in range(NSG):
                pltpu.sync_copy(src2.at[pl.ds(c0 + s * cps, cps)], gidx2)
                pltpu.sync_copy(dst2.at[pl.ds(c0 + s * cps, cps)], sidx2)
                pltpu.async_copy(yp.at[gidx2.at[0]], b0, sem0)
                pltpu.async_copy(yp.at[gidx2.at[1]], b1, sem1)

                def step(kk, _):
                    k0i = 2 * kk
                    pltpu.make_async_copy(yp.at[gidx2.at[0]], b0, sem0).wait()
                    pltpu.sync_copy(b0, acc.at[sidx2.at[k0i]], add=True)

                    @pl.when(kk < cps // 2 - 1)
                    def _():
                        pltpu.async_copy(yp.at[gidx2.at[k0i + 2]], b0, sem0)

                    pltpu.make_async_copy(yp.at[gidx2.at[1]], b1, sem1).wait()
                    pltpu.sync_copy(b1, acc.at[sidx2.at[k0i + 1]], add=True)

                    @pl.when(kk < cps // 2 - 1)
                    def _():
                        pltpu.async_copy(yp.at[gidx2.at[k0i + 3]], b1, sem1)

                    return 0

                lax.fori_loop(0, cps // 2, step, 0)

        @pl.when(cid == 0)
        def _():
            run(yp0)

        @pl.when(cid == 1)
        def _():
            run(yp1)

        plsc.subcore_barrier()

        def wout(o):
            pltpu.sync_copy(acc.at[pl.ds(r0, RPT)], o.at[pl.ds(r0, RPT)])

            @pl.when(sid == NS - 1)
            def _():
                pltpu.sync_copy(acc.at[pl.ds(REM0 + RPT, REM - RPT)],
                                o.at[pl.ds(REM0 + RPT, REM - RPT)])

        @pl.when(cid == 0)
        def _():
            wout(out0)

        @pl.when(cid == 1)
        def _():
            wout(out1)

    return pl.kernel(
        body,
        out_type=(jax.ShapeDtypeStruct((N, HH), jnp.float32),
                  jax.ShapeDtypeStruct((N, HH), jnp.float32)),
        mesh=mesh,
        scratch_types=[
            pltpu.VMEM((cps, CH), jnp.int32),
            pltpu.VMEM((cps, CH), jnp.int32),
            pltpu.VMEM((2, CH, HH), jnp.float32),
            pltpu.VMEM_SHARED((NPAD, HH), jnp.float32),
            pltpu.SemaphoreType.DMA,
            pltpu.SemaphoreType.DMA,
        ],
    )


@functools.lru_cache(maxsize=None)
def _kdeg(N, NW, ntk):
    def body(deg_r, dis_o):
        d = jnp.sum(deg_r[...], axis=0) + 1.0
        dis = lax.rsqrt(d).reshape(NW, 1)[:N]
        dis_o[...] = jnp.broadcast_to(dis, (N, 8))

    return pl.pallas_call(
        body,
        grid=(1,),
        in_specs=[pl.BlockSpec((ntk, NW), lambda i: (0, 0))],
        out_specs=pl.BlockSpec((N, 8), lambda i: (0, 0)),
        out_shape=jax.ShapeDtypeStruct((N, 8), jnp.float32),
    )


def _dot(a, b):
    return jnp.dot(a, b, preferred_element_type=jnp.float32)


def _gru(xi, hb, WihT, WhhT, bih, bhh, H):
    gi = _dot(xi, WihT) + bih
    gh = _dot(hb, WhhT) + bhh
    r = jax.nn.sigmoid(gi[:, :H] + gh[:, :H])
    z = jax.nn.sigmoid(gi[:, H:2 * H] + gh[:, H:2 * H])
    n = jnp.tanh(gi[:, 2 * H:] + r * gh[:, 2 * H:])
    return (1.0 - z) * n + z * hb


def _full(shape):
    return pl.BlockSpec(shape, lambda i: tuple(0 for _ in shape))


def _rows(bN, w):
    return pl.BlockSpec((bN, w), lambda i: (i, 0))


@functools.lru_cache(maxsize=None)
def _k0(N, F, H, bN):
    def body(dis_r, x_r, W_r, WihT_r, bih_r, bhh_r, h_o, y0_o, y1_o):
        dis = dis_r[...][:, :1]
        xb = x_r[...]
        yp = dis * _dot(xb, W_r[...])
        y0_o[...] = yp[:, :HH]
        y1_o[...] = yp[:, HH:]
        gi = _dot(xb, WihT_r[...]) + bih_r[...]
        bhh = bhh_r[...]
        r = jax.nn.sigmoid(gi[:, :H] + bhh[:, :H])
        z = jax.nn.sigmoid(gi[:, H:2 * H] + bhh[:, H:2 * H])
        n = jnp.tanh(gi[:, 2 * H:] + r * bhh[:, 2 * H:])
        h_o[...] = (1.0 - z) * n

    return pl.pallas_call(
        body,
        grid=(N // bN,),
        in_specs=[
            _rows(bN, 8),
            _rows(bN, F),
            _full((F, H)),
            _full((F, 3 * H)),
            _full((1, 3 * H)),
            _full((1, 3 * H)),
        ],
        out_specs=[_rows(bN, H), _rows(bN, HH), _rows(bN, HH)],
        out_shape=[
            jax.ShapeDtypeStruct((N, H), jnp.float32),
            jax.ShapeDtypeStruct((N, HH), jnp.float32),
            jax.ShapeDtypeStruct((N, HH), jnp.float32),
        ],
    )


@functools.lru_cache(maxsize=None)
def _kmid(N, H, bN):
    def body(dis_r, a0_r, a1_r, y0_r, y1_r, h_r, cb_r,
             WihT_r, WhhT_r, bih_r, bhh_r, Wn_r, h_o, y0_o, y1_o):
        dis = dis_r[...][:, :1]
        agg = jnp.concatenate([a0_r[...] + y0_r[...], a1_r[...] + y1_r[...]],
                              axis=1)
        xi = jnp.maximum(dis * agg + cb_r[...], 0.0)
        hb = h_r[...]
        hn = _gru(xi, hb, WihT_r[...], WhhT_r[...], bih_r[...], bhh_r[...], H)
        h_o[...] = hn
        ypn = dis * _dot(xi, Wn_r[...])
        y0_o[...] = ypn[:, :HH]
        y1_o[...] = ypn[:, HH:]

    return pl.pallas_call(
        body,
        grid=(N // bN,),
        in_specs=[
            _rows(bN, 8),
            _rows(bN, HH), _rows(bN, HH), _rows(bN, HH), _rows(bN, HH),
            _rows(bN, H),
            _full((1, H)),
            _full((H, 3 * H)), _full((H, 3 * H)),
            _full((1, 3 * H)), _full((1, 3 * H)),
            _full((H, H)),
        ],
        out_specs=[_rows(bN, H), _rows(bN, HH), _rows(bN, HH)],
        out_shape=[
            jax.ShapeDtypeStruct((N, H), jnp.float32),
            jax.ShapeDtypeStruct((N, HH), jnp.float32),
            jax.ShapeDtypeStruct((N, HH), jnp.float32),
        ],
    )


@functools.lru_cache(maxsize=None)
def _kfin(N, H, bN, CP):
    def body(dis_r, a0_r, a1_r, y0_r, y1_r, h_r, cb_r,
             WihT_r, WhhT_r, bih_r, bhh_r, fcW_r, fcb_r, o_r):
        dis = dis_r[...][:, :1]
        agg = jnp.concatenate([a0_r[...] + y0_r[...], a1_r[...] + y1_r[...]],
                              axis=1)
        xi = jnp.maximum(dis * agg + cb_r[...], 0.0)
        hb = h_r[...]
        hn = _gru(xi, hb, WihT_r[...], WhhT_r[...], bih_r[...], bhh_r[...], H)
        o_r[...] = _dot(hn, fcW_r[...]) + fcb_r[...]

    return pl.pallas_call(
        body,
        grid=(N // bN,),
        in_specs=[
            _rows(bN, 8),
            _rows(bN, HH), _rows(bN, HH), _rows(bN, HH), _rows(bN, HH),
            _rows(bN, H),
            _full((1, H)),
            _full((H, 3 * H)), _full((H, 3 * H)),
            _full((1, 3 * H)), _full((1, 3 * H)),
            _full((H, CP)), _full((1, CP)),
        ],
        out_specs=_rows(bN, CP),
        out_shape=jax.ShapeDtypeStruct((N, CP), jnp.float32),
    )


def kernel(x, edge_index, conv_W, conv_b, gru_Wih, gru_Whh, gru_bih, gru_bhh,
           fc_W, fc_b):
    N, F = x.shape
    E = edge_index.shape[1]
    H = gru_Whh.shape[-1] if gru_Whh.ndim == 3 else 256
    H = gru_Whh.shape[2]
    C = fc_W.shape[0]
    bN = 2000

    unit = CH * NS * NC
    EP = ((E + unit - 1) // unit) * unit
    pad = EP - E
    src = edge_index[0]
    dst = edge_index[1]
    if pad:
        src = jnp.concatenate([src, jnp.zeros((pad,), jnp.int32)])
        dst = jnp.concatenate([dst, jnp.full((pad,), N, jnp.int32)])
    src2 = src.reshape(EP // CH, CH)
    dst2 = dst.reshape(EP // CH, CH)

    NW = -(-N // 128) * 128
    deg = _deg_kernel(N, EP)(dst2).reshape(NC * NS, NW)
    dis = _kdeg(N, NW, NC * NS)(deg)

    WihT = jnp.swapaxes(gru_Wih, 1, 2)
    WhhT = jnp.swapaxes(gru_Whh, 1, 2)
    CP = 128
    fcW = jnp.zeros((H, CP), jnp.float32).at[:, :C].set(fc_W.T)
    fcb = jnp.zeros((1, CP), jnp.float32).at[0, :C].set(fc_b)

    h, y0, y1 = _k0(N, F, H, bN)(
        dis, x, conv_W[0], WihT[0], gru_bih[0][None], gru_bhh[0][None])

    scat = _scatter_kernel(N, EP)
    for i in range(1, 4):
        a0, a1 = scat(y0, y1, src2, dst2)
        h, y0, y1 = _kmid(N, H, bN)(
            dis, a0, a1, y0, y1, h, conv_b[i - 1][None],
            WihT[i], WhhT[i], gru_bih[i][None], gru_bhh[i][None], conv_W[i])
    a0, a1 = scat(y0, y1, src2, dst2)
    out = _kfin(N, H, bN, CP)(
        dis, a0, a1, y0, y1, h, conv_b[3][None],
        WihT[4], WhhT[4], gru_bih[4][None], gru_bhh[4][None], fcW, fcb)
    return out[:, :C]


# speedup vs baseline: 1.0555x; 1.0555x over previous
import functools

import jax
import jax.numpy as jnp
from jax import lax
from jax.experimental import pallas as pl
from jax.experimental.pallas import tpu as pltpu
from jax.experimental.pallas import tpu_sc as plsc

NC = 2
NS = 16
LN = 16
CH = 128
HH = 128
ZR = 128


def _fill(buf, rows, value):
    dt = jnp.dtype(buf.dtype)
    lanes = LN * (4 // dt.itemsize)
    ncol = buf.shape[1] // lanes
    v = jnp.full((lanes,), value, dt)
    for r in range(rows):
        for j in range(ncol):
            buf[r, pl.ds(j * lanes, lanes)] = v


def _row_split(N):
    RPT = (N // NS) // 8 * 8
    REM0 = RPT * (NS - 1)
    REM = N - REM0
    ZN = -(-max(RPT, REM) // ZR)
    assert REM0 + ZN * ZR <= N + 8 and REM >= RPT and (REM - RPT) % 8 == 0
    return RPT, REM0, REM, ZN


@functools.lru_cache(maxsize=None)
def _deg_kernel(N, EP):
    nch = EP // CH
    npt = nch // (NC * NS)
    NW = -(-N // 128) * 128
    NPAD = NW + 2 * LN
    mesh = plsc.VectorSubcoreMesh(core_axis_name="c", subcore_axis_name="s")
    ntk = NC * NS
    ne = npt * CH

    NI = 4

    def body(dst2_hbm, out_hbm, idx2, h0, h1, h2, h3):
        hists = (h0, h1, h2, h3)
        cid = lax.axis_index("c")
        sid = lax.axis_index("s")
        tid = cid * NS + sid
        pltpu.sync_copy(dst2_hbm.at[pl.ds(tid * npt, npt)], idx2)

        def zrow(r, _):
            for q in range(NI):
                hists[q][0, pl.ds(r * LN, LN)] = jnp.zeros((LN,), jnp.float32)
            return 0

        lax.fori_loop(0, NPAD // LN, zrow, 0)

        e0 = jnp.where(jnp.arange(LN, dtype=jnp.int32) == 0,
                       jnp.float32(1.0), jnp.float32(0.0))

        def chunk(k, _):
            for j in range(CH // LN):
                v = idx2[k, pl.ds(j * LN, LN)]
                for l in range(LN):
                    d = v[l]
                    hb = hists[l % NI]
                    w = hb[0, pl.ds(d, LN)]
                    hb[0, pl.ds(d, LN)] = w + e0
            return 0

        lax.fori_loop(0, npt, chunk, 0)
        for q in range(NI):
            pltpu.sync_copy(hists[q].at[:, pl.ds(0, NW)],
                            out_hbm.at[tid * NI + q])

    return pl.kernel(
        body,
        out_type=jax.ShapeDtypeStruct((ntk * NI, 1, NW), jnp.float32),
        mesh=mesh,
        scratch_types=[
            pltpu.VMEM((npt, CH), jnp.int32),
            pltpu.VMEM((1, NPAD), jnp.float32),
            pltpu.VMEM((1, NPAD), jnp.float32),
            pltpu.VMEM((1, NPAD), jnp.float32),
            pltpu.VMEM((1, NPAD), jnp.float32),
        ],
    )


@functools.lru_cache(maxsize=None)
def _scatter_kernel(N, EP):
    nch = EP // CH
    npt = nch // NS
    RPT, REM0, REM, ZN = _row_split(N)
    NPAD = N + 8
    mesh = plsc.VectorSubcoreMesh(core_axis_name="c", subcore_axis_name="s")

    NSG = 2
    cps = npt // NSG
    assert npt % NSG == 0 and cps % 2 == 0 and cps % 8 == 0

    def body(yp0, yp1, src2, dst2, out0, out1,
             gidx2, sidx2, bufs, acc, sem0, sem1):
        cid = lax.axis_index("c")
        sid = lax.axis_index("s")
        b0 = bufs.at[0]
        b1 = bufs.at[1]
        _fill(b0, ZR, 0.0)
        r0 = pl.multiple_of(sid * RPT, 8)
        for j in range(ZN):
            pltpu.sync_copy(b0, acc.at[pl.ds(r0 + j * ZR, ZR)])
        c0 = sid * npt
        plsc.subcore_barrier()

        def run(yp):
            for s in range(NSG):
                pltpu.sync_copy(src2.at[pl.ds(c0 + s * cps, cps)], gidx2)
                pltpu.sync_copy(dst2.at[pl.ds(c0 + s * cps, cps)], sidx2)
                pltpu.async_copy(yp.at[gidx2.at[0]], b0, sem0)
                pltpu.async_copy(yp.at[gidx2.at[1]], b1, sem1)

                def step(kk, _):
                    k0i = 2 * kk
                    pltpu.make_async_copy(yp.at[gidx2.at[0]], b0, sem0).wait()
                    pltpu.sync_copy(b0, acc.at[sidx2.at[k0i]], add=True)

                    @pl.when(kk < cps // 2 - 1)
                    def _():
                        pltpu.async_copy(yp.at[gidx2.at[k0i + 2]], b0, sem0)

                    pltpu.make_async_copy(yp.at[gidx2.at[1]], b1, sem1).wait()
                    pltpu.sync_copy(b1, acc.at[sidx2.at[k0i + 1]], add=True)

                    @pl.when(kk < cps // 2 - 1)
                    def _():
                        pltpu.async_copy(yp.at[gidx2.at[k0i + 3]], b1, sem1)

                    return 0

                lax.fori_loop(0, cps // 2, step, 0)

        @pl.when(cid == 0)
        def _():
            run(yp0)

        @pl.when(cid == 1)
        def _():
            run(yp1)

        plsc.subcore_barrier()

        def wout(o):
            pltpu.sync_copy(acc.at[pl.ds(r0, RPT)], o.at[pl.ds(r0, RPT)])

            @pl.when(sid == NS - 1)
            def _():
                pltpu.sync_copy(acc.at[pl.ds(REM0 + RPT, REM - RPT)],
                                o.at[pl.ds(REM0 + RPT, REM - RPT)])

        @pl.when(cid == 0)
        def _():
            wout(out0)

        @pl.when(cid == 1)
        def _():
            wout(out1)

    return pl.kernel(
        body,
        out_type=(jax.ShapeDtypeStruct((N, HH), jnp.float32),
                  jax.ShapeDtypeStruct((N, HH), jnp.float32)),
        mesh=mesh,
        scratch_types=[
            pltpu.VMEM((cps, CH), jnp.int32),
            pltpu.VMEM((cps, CH), jnp.int32),
            pltpu.VMEM((2, CH, HH), jnp.float32),
            pltpu.VMEM_SHARED((NPAD, HH), jnp.float32),
            pltpu.SemaphoreType.DMA,
            pltpu.SemaphoreType.DMA,
        ],
    )


@functools.lru_cache(maxsize=None)
def _kdeg(N, NW, ntk):
    def body(deg_r, dis_o):
        d = jnp.sum(deg_r[...], axis=0) + 1.0
        dis = lax.rsqrt(d).reshape(NW, 1)[:N]
        dis_o[...] = jnp.broadcast_to(dis, (N, 8))

    return pl.pallas_call(
        body,
        grid=(1,),
        in_specs=[pl.BlockSpec((ntk, NW), lambda i: (0, 0))],
        out_specs=pl.BlockSpec((N, 8), lambda i: (0, 0)),
        out_shape=jax.ShapeDtypeStruct((N, 8), jnp.float32),
    )


def _dot(a, b):
    return jnp.dot(a, b, preferred_element_type=jnp.float32)


def _gru(xi, hb, WihT, WhhT, bih, bhh, H):
    gi = _dot(xi, WihT) + bih
    gh = _dot(hb, WhhT) + bhh
    r = jax.nn.sigmoid(gi[:, :H] + gh[:, :H])
    z = jax.nn.sigmoid(gi[:, H:2 * H] + gh[:, H:2 * H])
    n = jnp.tanh(gi[:, 2 * H:] + r * gh[:, 2 * H:])
    return (1.0 - z) * n + z * hb


def _full(shape):
    return pl.BlockSpec(shape, lambda i: tuple(0 for _ in shape))


def _rows(bN, w):
    return pl.BlockSpec((bN, w), lambda i: (i, 0))


@functools.lru_cache(maxsize=None)
def _k0(N, F, H, bN):
    def body(dis_r, x_r, W_r, WihT_r, bih_r, bhh_r, h_o, y0_o, y1_o):
        dis = dis_r[...][:, :1]
        xb = x_r[...]
        yp = dis * _dot(xb, W_r[...])
        y0_o[...] = yp[:, :HH]
        y1_o[...] = yp[:, HH:]
        gi = _dot(xb, WihT_r[...]) + bih_r[...]
        bhh = bhh_r[...]
        r = jax.nn.sigmoid(gi[:, :H] + bhh[:, :H])
        z = jax.nn.sigmoid(gi[:, H:2 * H] + bhh[:, H:2 * H])
        n = jnp.tanh(gi[:, 2 * H:] + r * bhh[:, 2 * H:])
        h_o[...] = (1.0 - z) * n

    return pl.pallas_call(
        body,
        grid=(N // bN,),
        in_specs=[
            _rows(bN, 8),
            _rows(bN, F),
            _full((F, H)),
            _full((F, 3 * H)),
            _full((1, 3 * H)),
            _full((1, 3 * H)),
        ],
        out_specs=[_rows(bN, H), _rows(bN, HH), _rows(bN, HH)],
        out_shape=[
            jax.ShapeDtypeStruct((N, H), jnp.float32),
            jax.ShapeDtypeStruct((N, HH), jnp.float32),
            jax.ShapeDtypeStruct((N, HH), jnp.float32),
        ],
    )


@functools.lru_cache(maxsize=None)
def _kmid(N, H, bN):
    def body(dis_r, a0_r, a1_r, y0_r, y1_r, h_r, cb_r,
             WihT_r, WhhT_r, bih_r, bhh_r, Wn_r, h_o, y0_o, y1_o):
        dis = dis_r[...][:, :1]
        agg = jnp.concatenate([a0_r[...] + y0_r[...], a1_r[...] + y1_r[...]],
                              axis=1)
        xi = jnp.maximum(dis * agg + cb_r[...], 0.0)
        hb = h_r[...]
        hn = _gru(xi, hb, WihT_r[...], WhhT_r[...], bih_r[...], bhh_r[...], H)
        h_o[...] = hn
        ypn = dis * _dot(xi, Wn_r[...])
        y0_o[...] = ypn[:, :HH]
        y1_o[...] = ypn[:, HH:]

    return pl.pallas_call(
        body,
        grid=(N // bN,),
        in_specs=[
            _rows(bN, 8),
            _rows(bN, HH), _rows(bN, HH), _rows(bN, HH), _rows(bN, HH),
            _rows(bN, H),
            _full((1, H)),
            _full((H, 3 * H)), _full((H, 3 * H)),
            _full((1, 3 * H)), _full((1, 3 * H)),
            _full((H, H)),
        ],
        out_specs=[_rows(bN, H), _rows(bN, HH), _rows(bN, HH)],
        out_shape=[
            jax.ShapeDtypeStruct((N, H), jnp.float32),
            jax.ShapeDtypeStruct((N, HH), jnp.float32),
            jax.ShapeDtypeStruct((N, HH), jnp.float32),
        ],
    )


@functools.lru_cache(maxsize=None)
def _kfin(N, H, bN, CP):
    def body(dis_r, a0_r, a1_r, y0_r, y1_r, h_r, cb_r,
             WihT_r, WhhT_r, bih_r, bhh_r, fcW_r, fcb_r, o_r):
        dis = dis_r[...][:, :1]
        agg = jnp.concatenate([a0_r[...] + y0_r[...], a1_r[...] + y1_r[...]],
                              axis=1)
        xi = jnp.maximum(dis * agg + cb_r[...], 0.0)
        hb = h_r[...]
        hn = _gru(xi, hb, WihT_r[...], WhhT_r[...], bih_r[...], bhh_r[...], H)
        o_r[...] = _dot(hn, fcW_r[...]) + fcb_r[...]

    return pl.pallas_call(
        body,
        grid=(N // bN,),
        in_specs=[
            _rows(bN, 8),
            _rows(bN, HH), _rows(bN, HH), _rows(bN, HH), _rows(bN, HH),
            _rows(bN, H),
            _full((1, H)),
            _full((H, 3 * H)), _full((H, 3 * H)),
            _full((1, 3 * H)), _full((1, 3 * H)),
            _full((H, CP)), _full((1, CP)),
        ],
        out_specs=_rows(bN, CP),
        out_shape=jax.ShapeDtypeStruct((N, CP), jnp.float32),
    )


def kernel(x, edge_index, conv_W, conv_b, gru_Wih, gru_Whh, gru_bih, gru_bhh,
           fc_W, fc_b):
    N, F = x.shape
    E = edge_index.shape[1]
    H = gru_Whh.shape[-1] if gru_Whh.ndim == 3 else 256
    H = gru_Whh.shape[2]
    C = fc_W.shape[0]
    bN = 2000

    unit = CH * NS * NC
    EP = ((E + unit - 1) // unit) * unit
    pad = EP - E
    src = edge_index[0]
    dst = edge_index[1]
    if pad:
        src = jnp.concatenate([src, jnp.zeros((pad,), jnp.int32)])
        dst = jnp.concatenate([dst, jnp.full((pad,), N, jnp.int32)])
    src2 = src.reshape(EP // CH, CH)
    dst2 = dst.reshape(EP // CH, CH)

    NW = -(-N // 128) * 128
    deg = _deg_kernel(N, EP)(dst2).reshape(NC * NS * 4, NW)
    dis = _kdeg(N, NW, NC * NS * 4)(deg)

    WihT = jnp.swapaxes(gru_Wih, 1, 2)
    WhhT = jnp.swapaxes(gru_Whh, 1, 2)
    CP = 128
    fcW = jnp.zeros((H, CP), jnp.float32).at[:, :C].set(fc_W.T)
    fcb = jnp.zeros((1, CP), jnp.float32).at[0, :C].set(fc_b)

    h, y0, y1 = _k0(N, F, H, bN)(
        dis, x, conv_W[0], WihT[0], gru_bih[0][None], gru_bhh[0][None])

    scat = _scatter_kernel(N, EP)
    for i in range(1, 4):
        a0, a1 = scat(y0, y1, src2, dst2)
        h, y0, y1 = _kmid(N, H, bN)(
            dis, a0, a1, y0, y1, h, conv_b[i - 1][None],
            WihT[i], WhhT[i], gru_bih[i][None], gru_bhh[i][None], conv_W[i])
    a0, a1 = scat(y0, y1, src2, dst2)
    out = _kfin(N, H, bN, CP)(
        dis, a0, a1, y0, y1, h, conv_b[3][None],
        WihT[4], WhhT[4], gru_bih[4][None], gru_bhh[4][None], fcW, fcb)
    return out[:, :C]
